# baseline (speedup 1.0000x reference)
import functools

import jax
import jax.numpy as jnp
from jax import lax
from jax.experimental import pallas as pl
from jax.experimental.pallas import tpu as pltpu
from jax.experimental.pallas import tpu_sc as plsc

N = 10000
NPAD = 10240
D = 128
E = 320000
NSLICE = 4
DS = 32
NT = NPAD // 16
EC = 400
NEC = E // 16 // EC
DC = 2000
NDC = E // 16 // DC
SUB = 320
F32 = jnp.float32
I32 = jnp.int32


def _sc_body(xs, e0r, e0c, e1r, e1c, e2r, e2c, wts,
             out, h1h, dgh,
             ACC, SUM,
             dgp, iA0, iB0, iA1, iB1, idx2k,
             gA0, gB0, gA1, gB1,
             nt0, nt1, nt2, tst,
             wv, sA0, sB0, sA1, sB1, sS0, sS1):
    c = lax.axis_index("c")
    s = lax.axis_index("s")
    nb = s * NT
    ebase = s * (E // 16)

    zv = jnp.zeros((16,), F32)
    ones16 = jnp.full((16,), 1.0, F32)
    iAb = (iA0, iA1)
    iBb = (iB0, iB1)
    gAb = (gA0, gA1)
    gBb = (gB0, gB1)
    sAb = (sA0, sA1)
    sBb = (sB0, sB1)
    sSb = (sS0, sS1)

    def vfill(ref, n16, val):
        def zb(i, carry):
            ref[pl.ds(i * 16, 16)] = val
            return carry
        lax.fori_loop(0, n16, zb, 0)

    def zero_ACC():
        def zb(i, carry):
            gB1[i, pl.ds(0, 16)] = zv
            gB1[i, pl.ds(16, 16)] = zv
            return carry
        lax.fori_loop(0, SUB, zb, 0)
        pltpu.sync_copy(gB1.at[pl.ds(0, SUB)], ACC.at[pl.ds(nb, SUB)])
        pltpu.sync_copy(gB1.at[pl.ds(0, SUB)], ACC.at[pl.ds(nb + SUB, SUB)])

    def rel_apply(er, ec, src_hbm):
        def load_idx(b, ci):
            st = ebase + ci * EC
            pltpu.sync_copy(er.at[pl.ds(st, EC)], iAb[b])
            pltpu.sync_copy(ec.at[pl.ds(st, EC)], iBb[b])
        def fire(b):
            pltpu.async_copy(src_hbm.at[iAb[b]], gAb[b], sAb[b])
            pltpu.async_copy(src_hbm.at[iBb[b]], gBb[b], sBb[b])
        def drain(b):
            pltpu.make_async_copy(src_hbm.at[iAb[b]], gAb[b], sAb[b]).wait()
            pltpu.make_async_copy(src_hbm.at[iBb[b]], gBb[b], sBb[b]).wait()
        def fire_s(b):
            d1 = pltpu.async_copy(gAb[b], ACC.at[iBb[b]], sSb[b], add=True)
            d2 = pltpu.async_copy(gBb[b], ACC.at[iAb[b]], sSb[b], add=True)
            return d1, d2

        load_idx(0, 0)
        fire(0)
        def body(h, carry):
            load_idx(1, 2 * h + 1)
            fire(1)
            drain(0)
            s0 = fire_s(0)
            drain(1)
            s1 = fire_s(1)
            s0[0].wait()
            s0[1].wait()
            @pl.when(h + 1 < NEC // 2)
            def _():
                load_idx(0, 2 * h + 2)
                fire(0)
            s1[0].wait()
            s1[1].wait()
            return carry
        lax.fori_loop(0, NEC // 2, body, 0)

    def deg_round(er, ec, ntr):
        vfill(dgp, NPAD // 16, zv)
        def inner(i, carry):
            iv = idx2k[pl.ds(i * 16, 16)]
            plsc.addupdate_scatter(dgp, [iv], ones16)
            return carry
        def body(ci, carry):
            st = ebase + ci * DC
            pltpu.sync_copy(er.at[pl.ds(st, DC)], idx2k)
            lax.fori_loop(0, DC // 16, inner, 0)
            pltpu.sync_copy(ec.at[pl.ds(st, DC)], idx2k)
            lax.fori_loop(0, DC // 16, inner, 0)
            return carry
        lax.fori_loop(0, NDC, body, 0)
        pltpu.sync_copy(dgp, dgh.at[c, s])
        plsc.subcore_barrier()
        vfill(ntr, NT // 16, zv)
        def trow(t, carry):
            pltpu.sync_copy(dgh.at[c, t, pl.ds(nb, NT)], tst)
            def addv(i, c2):
                sl = pl.ds(i * 16, 16)
                ntr[sl] = ntr[sl] + tst[sl]
                return c2
            lax.fori_loop(0, NT // 16, addv, 0)
            return carry
        lax.fori_loop(0, 16, trow, 0)
        plsc.subcore_barrier()

    deg_round(e0r, e0c, nt0)
    deg_round(e1r, e1c, nt1)
    deg_round(e2r, e2c, nt2)

    def norm_body(i, carry):
        sl = pl.ds(i * 16, 16)
        d0 = nt0[sl]
        d1 = nt1[sl]
        d2 = nt2[sl]
        dsum = d0 + d1 + d2
        nt0[sl] = jnp.where(d0 > 0, 1.0 / d0, 0.0)
        nt1[sl] = jnp.where(d1 > 0, 1.0 / d1, 0.0)
        nt2[sl] = jnp.where(dsum > 0, 1.0 / dsum, 0.0)
        return carry
    lax.fori_loop(0, NT // 16, norm_body, 0)

    P0, P1, R, R2 = gA0, gA1, gB0, gB1

    def splat(ntr, nl):
        return plsc.load_gather(ntr, [jnp.full((16,), nl, I32)])

    def pass_body(p, carry):
        k = 2 * c + p
        pltpu.sync_copy(wts.at[k], wv)

        def hop1_round(er, ec, ntr, r):
            zero_ACC()
            plsc.subcore_barrier()
            rel_apply(er, ec, xs.at[k])
            plsc.subcore_barrier()
            def vsub(sub, carry2):
                base = nb + sub * SUB
                pltpu.sync_copy(ACC.at[pl.ds(base, SUB)], P0.at[pl.ds(0, SUB)])
                if r == 0:
                    pltpu.sync_copy(P0.at[pl.ds(0, SUB)], SUM.at[pl.ds(base, SUB)])
                else:
                    pltpu.sync_copy(SUM.at[pl.ds(base, SUB)], P1.at[pl.ds(0, SUB)])
                def addb(n, c3):
                    for j in range(2):
                        sl = pl.ds(j * 16, 16)
                        P1[n, sl] = P1[n, sl] + P0[n, sl]
                    return c3
                if r != 0:
                    lax.fori_loop(0, SUB, addb, 0)
                    pltpu.sync_copy(P1.at[pl.ds(0, SUB)], SUM.at[pl.ds(base, SUB)])
                srcv = P1 if r == 2 else P0
                def h1b(n, c3):
                    nv = splat(ntr, sub * SUB + n)
                    for j in range(2):
                        sl = pl.ds(j * 16, 16)
                        R[n, sl] = srcv[n, sl] * nv
                    return c3
                lax.fori_loop(0, SUB, h1b, 0)
                pltpu.sync_copy(R.at[pl.ds(0, SUB)], h1h.at[r, k, pl.ds(base, SUB)])
                return carry2
            lax.fori_loop(0, 2, vsub, 0)
            plsc.subcore_barrier()

        hop1_round(e0r, e0c, nt0, 0)
        hop1_round(e1r, e1c, nt1, 1)
        hop1_round(e2r, e2c, nt2, 2)

        def hop2_round(ntr, j):
            zero_ACC()
            plsc.subcore_barrier()
            if j == 0:
                rel_apply(e0r, e0c, h1h.at[0, k])
            elif j == 1:
                rel_apply(e1r, e1c, h1h.at[1, k])
            else:
                rel_apply(e0r, e0c, h1h.at[2, k])
                rel_apply(e1r, e1c, h1h.at[2, k])
                rel_apply(e2r, e2c, h1h.at[2, k])
            plsc.subcore_barrier()
            wa = wv[4 + j, pl.ds(0, 16)]
            wb = wv[4 + j, pl.ds(16, 16)]
            def vsub(sub, carry2):
                base = nb + sub * SUB
                pltpu.sync_copy(ACC.at[pl.ds(base, SUB)], P0.at[pl.ds(0, SUB)])
                if j > 0:
                    pltpu.sync_copy(SUM.at[pl.ds(base, SUB)], P1.at[pl.ds(0, SUB)])
                def nbody(n, c3):
                    nv = splat(ntr, sub * SUB + n)
                    fa = P0[n, pl.ds(0, 16)] * nv * wa
                    fb = P0[n, pl.ds(16, 16)] * nv * wb
                    if j > 0:
                        fa = fa + P1[n, pl.ds(0, 16)]
                        fb = fb + P1[n, pl.ds(16, 16)]
                    R[n, pl.ds(0, 16)] = fa
                    R[n, pl.ds(16, 16)] = fb
                    return c3
                lax.fori_loop(0, SUB, nbody, 0)
                if j == 2:
                    pltpu.sync_copy(R.at[pl.ds(0, SUB)],
                                    out.at[2, k, pl.ds(base, SUB)])
                else:
                    pltpu.sync_copy(R.at[pl.ds(0, SUB)],
                                    SUM.at[pl.ds(base, SUB)])
                return carry2
            lax.fori_loop(0, 2, vsub, 0)
            plsc.subcore_barrier()

        hop2_round(nt0, 0)
        hop2_round(nt1, 1)
        hop2_round(nt2, 2)

        w0a = wv[0, pl.ds(0, 16)]
        w0b = wv[0, pl.ds(16, 16)]
        w1 = [(wv[1 + r, pl.ds(0, 16)], wv[1 + r, pl.ds(16, 16)])
              for r in range(3)]
        def osub(sub, carry2):
            base = nb + sub * SUB
            pltpu.sync_copy(xs.at[k, pl.ds(base, SUB)], P0.at[pl.ds(0, SUB)])
            def o0(n, c3):
                R[n, pl.ds(0, 16)] = P0[n, pl.ds(0, 16)] * w0a
                R[n, pl.ds(16, 16)] = P0[n, pl.ds(16, 16)] * w0b
                return c3
            lax.fori_loop(0, SUB, o0, 0)
            pltpu.sync_copy(R.at[pl.ds(0, SUB)], out.at[0, k, pl.ds(base, SUB)])
            pltpu.sync_copy(h1h.at[0, k, pl.ds(base, SUB)], P0.at[pl.ds(0, SUB)])
            pltpu.sync_copy(h1h.at[1, k, pl.ds(base, SUB)], P1.at[pl.ds(0, SUB)])
            pltpu.sync_copy(h1h.at[2, k, pl.ds(base, SUB)], R2.at[pl.ds(0, SUB)])
            def o1(n, c3):
                for j in range(2):
                    sl = pl.ds(j * 16, 16)
                    R[n, sl] = (P0[n, sl] * w1[0][j]
                                + P1[n, sl] * w1[1][j]
                                + R2[n, sl] * w1[2][j])
                return c3
            lax.fori_loop(0, SUB, o1, 0)
            pltpu.sync_copy(R.at[pl.ds(0, SUB)], out.at[1, k, pl.ds(base, SUB)])
            return carry2
        lax.fori_loop(0, 2, osub, 0)
        plsc.subcore_barrier()
        return carry

    lax.fori_loop(0, 2, pass_body, 0)


_nars_sc = functools.partial(
    pl.kernel,
    out_type=(
        jax.ShapeDtypeStruct((3, NSLICE, NPAD, DS), F32),
        jax.ShapeDtypeStruct((3, NSLICE, NPAD, DS), F32),
        jax.ShapeDtypeStruct((2, 16, NPAD), F32),
    ),
    mesh=plsc.VectorSubcoreMesh(core_axis_name="c", subcore_axis_name="s"),
    compiler_params=pltpu.CompilerParams(
        needs_layout_passes=False, use_tc_tiling_on_sc=False),
    scratch_types=[
        pltpu.VMEM_SHARED((NPAD, DS), F32),
        pltpu.VMEM_SHARED((NPAD, DS), F32),
        pltpu.VMEM((NPAD,), F32),
        pltpu.VMEM((EC,), I32),
        pltpu.VMEM((EC,), I32),
        pltpu.VMEM((EC,), I32),
        pltpu.VMEM((EC,), I32),
        pltpu.VMEM((DC,), I32),
        pltpu.VMEM((EC, DS), F32),
        pltpu.VMEM((EC, DS), F32),
        pltpu.VMEM((EC, DS), F32),
        pltpu.VMEM((EC, DS), F32),
        pltpu.VMEM((NT,), F32),
        pltpu.VMEM((NT,), F32),
        pltpu.VMEM((NT,), F32),
        pltpu.VMEM((NT,), F32),
        pltpu.VMEM((8, DS), F32),
        pltpu.SemaphoreType.DMA,
        pltpu.SemaphoreType.DMA,
        pltpu.SemaphoreType.DMA,
        pltpu.SemaphoreType.DMA,
        pltpu.SemaphoreType.DMA,
        pltpu.SemaphoreType.DMA,
    ],
)(_sc_body)


def kernel(x, edge_index_r0, edge_index_r1, edge_index_r2, weight):
    xp = jnp.pad(x, ((0, NPAD - N), (0, 0)))
    xs = xp.reshape(NPAD, NSLICE, DS).transpose(1, 0, 2)
    w = weight.reshape(3, 3, D)
    w0c = w[0].sum(axis=0)
    rows = jnp.concatenate([w0c[None], w[1], w[2], jnp.zeros((1, D), F32)],
                           axis=0)
    wts = rows.reshape(8, NSLICE, DS).transpose(1, 0, 2)
    out4, _, _ = _nars_sc(xs,
                          edge_index_r0[0], edge_index_r0[1],
                          edge_index_r1[0], edge_index_r1[1],
                          edge_index_r2[0], edge_index_r2[1], wts)
    out = out4.transpose(0, 2, 1, 3).reshape(3, NPAD, D)[:, :N]
    return out


# speedup vs baseline: 1.2372x; 1.2372x over previous
import functools

import jax
import jax.numpy as jnp
from jax import lax
from jax.experimental import pallas as pl
from jax.experimental.pallas import tpu as pltpu
from jax.experimental.pallas import tpu_sc as plsc

N = 10000
NPAD = 10240
D = 128
E = 320000
NSLICE = 4
DS = 32
NT = NPAD // 16
EC = 400
NEC = E // 16 // EC
DC = 2000
NDC = E // 16 // DC
SUB = 320
F32 = jnp.float32
I32 = jnp.int32


def _sc_body(xs, e0r, e0c, e1r, e1c, e2r, e2c, wts,
             out, h1h, dgh, uh,
             ACC,
             dgp, ialA, ialB,
             gA0, gB0, gA1, gB1,
             nt0, nt1, nt2, tst,
             wv, sA0, sB0, sA1, sB1):
    c = lax.axis_index("c")
    s = lax.axis_index("s")
    nb = s * NT
    ebase = s * (E // 16)

    zv = jnp.zeros((16,), F32)
    ones16 = jnp.full((16,), 1.0, F32)
    gAb = (gA0, gA1)
    gBb = (gB0, gB1)
    sAb = (sA0, sA1)
    sBb = (sB0, sB1)

    def vfill(ref, n16, val):
        def zb(i, carry):
            ref[pl.ds(i * 16, 16)] = val
            return carry
        lax.fori_loop(0, n16, zb, 0)

    def zero_ACC():
        def zb(i, carry):
            gB1[i, pl.ds(0, 16)] = zv
            gB1[i, pl.ds(16, 16)] = zv
            return carry
        lax.fori_loop(0, SUB, zb, 0)
        pltpu.sync_copy(gB1.at[pl.ds(0, SUB)], ACC.at[pl.ds(nb, SUB)])
        pltpu.sync_copy(gB1.at[pl.ds(0, SUB)], ACC.at[pl.ds(nb + SUB, SUB)])

    def rel_apply(er2, ec2, src_hbm):
        pltpu.sync_copy(er2.at[pl.ds(s * NEC, NEC)], ialA)
        pltpu.sync_copy(ec2.at[pl.ds(s * NEC, NEC)], ialB)
        def fire(b, ci):
            pltpu.async_copy(src_hbm.at[ialA.at[ci]], gAb[b], sAb[b])
            pltpu.async_copy(src_hbm.at[ialB.at[ci]], gBb[b], sBb[b])
        def drain(b, ci):
            pltpu.make_async_copy(src_hbm.at[ialA.at[ci]], gAb[b], sAb[b]).wait()
            pltpu.make_async_copy(src_hbm.at[ialB.at[ci]], gBb[b], sBb[b]).wait()
        def scat(b, ci):
            pltpu.sync_copy(gAb[b], ACC.at[ialB.at[ci]], add=True)
            pltpu.sync_copy(gBb[b], ACC.at[ialA.at[ci]], add=True)

        fire(0, 0)
        def body(h, carry):
            ci0 = 2 * h
            ci1 = 2 * h + 1
            fire(1, ci1)
            drain(0, ci0)
            scat(0, ci0)
            @pl.when(h + 1 < NEC // 2)
            def _():
                fire(0, ci0 + 2)
            drain(1, ci1)
            scat(1, ci1)
            return carry
        lax.fori_loop(0, NEC // 2, body, 0)

    def deg_round(er2, ec2, ntr):
        pltpu.sync_copy(er2.at[pl.ds(s * NEC, NEC)], ialA)
        pltpu.sync_copy(ec2.at[pl.ds(s * NEC, NEC)], ialB)
        vfill(dgp, NPAD // 16, zv)
        def body(ci, carry):
            def inner(i, c2):
                sl = pl.ds(i * 16, 16)
                plsc.addupdate_scatter(dgp, [ialA[ci, sl]], ones16)
                plsc.addupdate_scatter(dgp, [ialB[ci, sl]], ones16)
                return c2
            lax.fori_loop(0, EC // 16, inner, 0)
            return carry
        lax.fori_loop(0, NEC, body, 0)
        pltpu.sync_copy(dgp, dgh.at[c, s])
        plsc.subcore_barrier()
        vfill(ntr, NT // 16, zv)
        def trow(t, carry):
            pltpu.sync_copy(dgh.at[c, t, pl.ds(nb, NT)], tst)
            def addv(i, c2):
                sl = pl.ds(i * 16, 16)
                ntr[sl] = ntr[sl] + tst[sl]
                return c2
            lax.fori_loop(0, NT // 16, addv, 0)
            return carry
        lax.fori_loop(0, 16, trow, 0)
        plsc.subcore_barrier()

    deg_round(e0r, e0c, nt0)
    deg_round(e1r, e1c, nt1)
    deg_round(e2r, e2c, nt2)

    def norm_body(i, carry):
        sl = pl.ds(i * 16, 16)
        d0 = nt0[sl]
        d1 = nt1[sl]
        d2 = nt2[sl]
        dsum = d0 + d1 + d2
        nt0[sl] = jnp.where(d0 > 0, 1.0 / d0, 0.0)
        nt1[sl] = jnp.where(d1 > 0, 1.0 / d1, 0.0)
        nt2[sl] = jnp.where(dsum > 0, 1.0 / dsum, 0.0)
        return carry
    lax.fori_loop(0, NT // 16, norm_body, 0)

    P0, P1, R, R2 = gA0, gA1, gB0, gB1

    def splat(ntr, nl):
        return plsc.load_gather(ntr, [jnp.full((16,), nl, I32)])

    def pass_body(p, carry):
        k = 2 * c + p
        pltpu.sync_copy(wts.at[k], wv)

        def hop1_round(er, ec, ntr, r):
            zero_ACC()
            plsc.subcore_barrier()
            rel_apply(er, ec, xs.at[k])
            plsc.subcore_barrier()
            def vsub(sub, carry2):
                base = nb + sub * SUB
                pltpu.sync_copy(ACC.at[pl.ds(base, SUB)], P0.at[pl.ds(0, SUB)])
                if r == 0:
                    pltpu.sync_copy(P0.at[pl.ds(0, SUB)], uh.at[c, pl.ds(base, SUB)])
                else:
                    pltpu.sync_copy(uh.at[c, pl.ds(base, SUB)], P1.at[pl.ds(0, SUB)])
                def addb(n, c3):
                    for j in range(2):
                        sl = pl.ds(j * 16, 16)
                        P1[n, sl] = P1[n, sl] + P0[n, sl]
                    return c3
                if r != 0:
                    lax.fori_loop(0, SUB, addb, 0)
                if r == 1:
                    pltpu.sync_copy(P1.at[pl.ds(0, SUB)], uh.at[c, pl.ds(base, SUB)])
                srcv = P1 if r == 2 else P0
                def h1b(n, c3):
                    nv = splat(ntr, sub * SUB + n)
                    for j in range(2):
                        sl = pl.ds(j * 16, 16)
                        R[n, sl] = srcv[n, sl] * nv
                    return c3
                lax.fori_loop(0, SUB, h1b, 0)
                pltpu.sync_copy(R.at[pl.ds(0, SUB)], h1h.at[r, k, pl.ds(base, SUB)])
                return carry2
            lax.fori_loop(0, 2, vsub, 0)
            plsc.subcore_barrier()

        hop1_round(e0r, e0c, nt0, 0)
        hop1_round(e1r, e1c, nt1, 1)
        hop1_round(e2r, e2c, nt2, 2)

        def hop2_round(ntr, j):
            zero_ACC()
            plsc.subcore_barrier()
            if j == 0:
                rel_apply(e0r, e0c, h1h.at[0, k])
            elif j == 1:
                rel_apply(e1r, e1c, h1h.at[1, k])
            else:
                rel_apply(e0r, e0c, h1h.at[2, k])
                rel_apply(e1r, e1c, h1h.at[2, k])
                rel_apply(e2r, e2c, h1h.at[2, k])
            plsc.subcore_barrier()
            wa = wv[4 + j, pl.ds(0, 16)]
            wb = wv[4 + j, pl.ds(16, 16)]
            def vsub(sub, carry2):
                base = nb + sub * SUB
                pltpu.sync_copy(ACC.at[pl.ds(base, SUB)], P0.at[pl.ds(0, SUB)])
                if j > 0:
                    pltpu.sync_copy(uh.at[c, pl.ds(base, SUB)], P1.at[pl.ds(0, SUB)])
                def nbody(n, c3):
                    nv = splat(ntr, sub * SUB + n)
                    fa = P0[n, pl.ds(0, 16)] * nv * wa
                    fb = P0[n, pl.ds(16, 16)] * nv * wb
                    if j > 0:
                        fa = fa + P1[n, pl.ds(0, 16)]
                        fb = fb + P1[n, pl.ds(16, 16)]
                    R[n, pl.ds(0, 16)] = fa
                    R[n, pl.ds(16, 16)] = fb
                    return c3
                lax.fori_loop(0, SUB, nbody, 0)
                if j == 2:
                    pltpu.sync_copy(R.at[pl.ds(0, SUB)],
                                    out.at[2, k, pl.ds(base, SUB)])
                else:
                    pltpu.sync_copy(R.at[pl.ds(0, SUB)],
                                    uh.at[c, pl.ds(base, SUB)])
                return carry2
            lax.fori_loop(0, 2, vsub, 0)
            plsc.subcore_barrier()

        hop2_round(nt0, 0)
        hop2_round(nt1, 1)
        hop2_round(nt2, 2)

        w0a = wv[0, pl.ds(0, 16)]
        w0b = wv[0, pl.ds(16, 16)]
        w1 = [(wv[1 + r, pl.ds(0, 16)], wv[1 + r, pl.ds(16, 16)])
              for r in range(3)]
        def osub(sub, carry2):
            base = nb + sub * SUB
            pltpu.sync_copy(xs.at[k, pl.ds(base, SUB)], P0.at[pl.ds(0, SUB)])
            def o0(n, c3):
                R[n, pl.ds(0, 16)] = P0[n, pl.ds(0, 16)] * w0a
                R[n, pl.ds(16, 16)] = P0[n, pl.ds(16, 16)] * w0b
                return c3
            lax.fori_loop(0, SUB, o0, 0)
            pltpu.sync_copy(R.at[pl.ds(0, SUB)], out.at[0, k, pl.ds(base, SUB)])
            pltpu.sync_copy(h1h.at[0, k, pl.ds(base, SUB)], P0.at[pl.ds(0, SUB)])
            pltpu.sync_copy(h1h.at[1, k, pl.ds(base, SUB)], P1.at[pl.ds(0, SUB)])
            pltpu.sync_copy(h1h.at[2, k, pl.ds(base, SUB)], R2.at[pl.ds(0, SUB)])
            def o1(n, c3):
                for j in range(2):
                    sl = pl.ds(j * 16, 16)
                    R[n, sl] = (P0[n, sl] * w1[0][j]
                                + P1[n, sl] * w1[1][j]
                                + R2[n, sl] * w1[2][j])
                return c3
            lax.fori_loop(0, SUB, o1, 0)
            pltpu.sync_copy(R.at[pl.ds(0, SUB)], out.at[1, k, pl.ds(base, SUB)])
            return carry2
        lax.fori_loop(0, 2, osub, 0)
        plsc.subcore_barrier()
        return carry

    lax.fori_loop(0, 2, pass_body, 0)


_nars_sc = functools.partial(
    pl.kernel,
    out_type=(
        jax.ShapeDtypeStruct((3, NSLICE, NPAD, DS), F32),
        jax.ShapeDtypeStruct((3, NSLICE, NPAD, DS), F32),
        jax.ShapeDtypeStruct((2, 16, NPAD), F32),
        jax.ShapeDtypeStruct((2, NPAD, DS), F32),
    ),
    mesh=plsc.VectorSubcoreMesh(core_axis_name="c", subcore_axis_name="s"),
    compiler_params=pltpu.CompilerParams(
        needs_layout_passes=False, use_tc_tiling_on_sc=False),
    scratch_types=[
        pltpu.VMEM_SHARED((NPAD, DS), F32),
        pltpu.VMEM((NPAD,), F32),
        pltpu.VMEM((NEC, EC), I32),
        pltpu.VMEM((NEC, EC), I32),
        pltpu.VMEM((EC, DS), F32),
        pltpu.VMEM((EC, DS), F32),
        pltpu.VMEM((EC, DS), F32),
        pltpu.VMEM((EC, DS), F32),
        pltpu.VMEM((NT,), F32),
        pltpu.VMEM((NT,), F32),
        pltpu.VMEM((NT,), F32),
        pltpu.VMEM((NT,), F32),
        pltpu.VMEM((8, DS), F32),
        pltpu.SemaphoreType.DMA,
        pltpu.SemaphoreType.DMA,
        pltpu.SemaphoreType.DMA,
        pltpu.SemaphoreType.DMA,
    ],
)(_sc_body)


def kernel(x, edge_index_r0, edge_index_r1, edge_index_r2, weight):
    xp = jnp.pad(x, ((0, NPAD - N), (0, 0)))
    xs = xp.reshape(NPAD, NSLICE, DS).transpose(1, 0, 2)
    w = weight.reshape(3, 3, D)
    w0c = w[0].sum(axis=0)
    rows = jnp.concatenate([w0c[None], w[1], w[2], jnp.zeros((1, D), F32)],
                           axis=0)
    wts = rows.reshape(8, NSLICE, DS).transpose(1, 0, 2)
    e2d = [e[i].reshape(16 * NEC, EC)
           for e in (edge_index_r0, edge_index_r1, edge_index_r2)
           for i in (0, 1)]
    out4, _, _, _ = _nars_sc(xs, *e2d, wts)
    out = out4.transpose(0, 2, 1, 3).reshape(3, NPAD, D)[:, :N]
    return out


# speedup vs baseline: 1.3820x; 1.1171x over previous
import functools

import jax
import jax.numpy as jnp
from jax import lax
from jax.experimental import pallas as pl
from jax.experimental.pallas import tpu as pltpu
from jax.experimental.pallas import tpu_sc as plsc

N = 10000
NPAD = 10240
D = 128
E = 320000
NSLICE = 4
DS = 32
NT = NPAD // 16
EC = 200
NEC = E // 16 // EC
DC = 2000
NDC = E // 16 // DC
SUB = 160
F32 = jnp.float32
I32 = jnp.int32


def _sc_body(xs, e0r, e0c, e1r, e1c, e2r, e2c, wts,
             out, h1h, dgh, uh,
             ACC,
             dgp, ialA, ialB,
             gA0, gB0, gA1, gB1, gA2, gB2, gA3, gB3,
             nt0, nt1, nt2, tst,
             wv, sA0, sB0, sA1, sB1, sA2, sB2, sA3, sB3):
    c = lax.axis_index("c")
    s = lax.axis_index("s")
    nb = s * NT
    ebase = s * (E // 16)

    zv = jnp.zeros((16,), F32)
    ones16 = jnp.full((16,), 1.0, F32)
    gAb = (gA0, gA1, gA2, gA3)
    gBb = (gB0, gB1, gB2, gB3)
    sAb = (sA0, sA1, sA2, sA3)
    sBb = (sB0, sB1, sB2, sB3)

    def vfill(ref, n16, val):
        def zb(i, carry):
            ref[pl.ds(i * 16, 16)] = val
            return carry
        lax.fori_loop(0, n16, zb, 0)

    def zero_ACC():
        def zb(i, carry):
            gB1[i, pl.ds(0, 16)] = zv
            gB1[i, pl.ds(16, 16)] = zv
            return carry
        lax.fori_loop(0, SUB, zb, 0)
        for q in range(NT // SUB):
            pltpu.sync_copy(gB1.at[pl.ds(0, SUB)],
                            ACC.at[pl.ds(nb + q * SUB, SUB)])

    def rel_apply(er2, ec2, src_hbm):
        pltpu.sync_copy(er2.at[pl.ds(s * NEC, NEC)], ialA)
        pltpu.sync_copy(ec2.at[pl.ds(s * NEC, NEC)], ialB)
        def fire(b, ci):
            pltpu.async_copy(src_hbm.at[ialA.at[ci]], gAb[b], sAb[b])
            pltpu.async_copy(src_hbm.at[ialB.at[ci]], gBb[b], sBb[b])
        def drain(b, ci):
            pltpu.make_async_copy(src_hbm.at[ialA.at[ci]], gAb[b], sAb[b]).wait()
            pltpu.make_async_copy(src_hbm.at[ialB.at[ci]], gBb[b], sBb[b]).wait()
        def scat(b, ci):
            pltpu.sync_copy(gAb[b], ACC.at[ialB.at[ci]], add=True)
            pltpu.sync_copy(gBb[b], ACC.at[ialA.at[ci]], add=True)

        fire(0, 0)
        fire(1, 1)
        fire(2, 2)
        def body(h, carry):
            for b in range(4):
                ci = 4 * h + b
                nxt = ci + 3
                @pl.when(nxt < NEC)
                def _():
                    fire((b + 3) % 4, nxt)
                drain(b, ci)
                scat(b, ci)
            return carry
        lax.fori_loop(0, NEC // 4, body, 0)

    def deg_round(er2, ec2, ntr):
        pltpu.sync_copy(er2.at[pl.ds(s * NEC, NEC)], ialA)
        pltpu.sync_copy(ec2.at[pl.ds(s * NEC, NEC)], ialB)
        vfill(dgp, NPAD // 16, zv)
        def body(ci, carry):
            def inner(i, c2):
                sl = pl.ds(i * 16, 16)
                plsc.addupdate_scatter(dgp, [ialA[ci, sl]], ones16)
                plsc.addupdate_scatter(dgp, [ialB[ci, sl]], ones16)
                return c2
            lax.fori_loop(0, EC // 16, inner, 0)
            return carry
        lax.fori_loop(0, NEC, body, 0)
        pltpu.sync_copy(dgp, dgh.at[c, s])
        plsc.subcore_barrier()
        vfill(ntr, NT // 16, zv)
        def trow(t, carry):
            pltpu.sync_copy(dgh.at[c, t, pl.ds(nb, NT)], tst)
            def addv(i, c2):
                sl = pl.ds(i * 16, 16)
                ntr[sl] = ntr[sl] + tst[sl]
                return c2
            lax.fori_loop(0, NT // 16, addv, 0)
            return carry
        lax.fori_loop(0, 16, trow, 0)
        plsc.subcore_barrier()

    deg_round(e0r, e0c, nt0)
    deg_round(e1r, e1c, nt1)
    deg_round(e2r, e2c, nt2)

    def norm_body(i, carry):
        sl = pl.ds(i * 16, 16)
        d0 = nt0[sl]
        d1 = nt1[sl]
        d2 = nt2[sl]
        dsum = d0 + d1 + d2
        nt0[sl] = jnp.where(d0 > 0, 1.0 / d0, 0.0)
        nt1[sl] = jnp.where(d1 > 0, 1.0 / d1, 0.0)
        nt2[sl] = jnp.where(dsum > 0, 1.0 / dsum, 0.0)
        return carry
    lax.fori_loop(0, NT // 16, norm_body, 0)

    P0, P1, R, R2 = gA0, gA1, gB0, gB1

    def splat(ntr, nl):
        return plsc.load_gather(ntr, [jnp.full((16,), nl, I32)])

    def pass_body(p, carry):
        k = 2 * c + p
        pltpu.sync_copy(wts.at[k], wv)

        def hop1_round(er, ec, ntr, r):
            zero_ACC()
            plsc.subcore_barrier()
            rel_apply(er, ec, xs.at[k])
            plsc.subcore_barrier()
            def vsub(sub, carry2):
                base = nb + sub * SUB
                pltpu.sync_copy(ACC.at[pl.ds(base, SUB)], P0.at[pl.ds(0, SUB)])
                if r == 0:
                    pltpu.sync_copy(P0.at[pl.ds(0, SUB)], uh.at[c, pl.ds(base, SUB)])
                else:
                    pltpu.sync_copy(uh.at[c, pl.ds(base, SUB)], P1.at[pl.ds(0, SUB)])
                def addb(n, c3):
                    for j in range(2):
                        sl = pl.ds(j * 16, 16)
                        P1[n, sl] = P1[n, sl] + P0[n, sl]
                    return c3
                if r != 0:
                    lax.fori_loop(0, SUB, addb, 0)
                if r == 1:
                    pltpu.sync_copy(P1.at[pl.ds(0, SUB)], uh.at[c, pl.ds(base, SUB)])
                srcv = P1 if r == 2 else P0
                def h1b(n, c3):
                    nv = splat(ntr, sub * SUB + n)
                    for j in range(2):
                        sl = pl.ds(j * 16, 16)
                        R[n, sl] = srcv[n, sl] * nv
                    return c3
                lax.fori_loop(0, SUB, h1b, 0)
                pltpu.sync_copy(R.at[pl.ds(0, SUB)], h1h.at[r, k, pl.ds(base, SUB)])
                return carry2
            lax.fori_loop(0, NT // SUB, vsub, 0)
            plsc.subcore_barrier()

        hop1_round(e0r, e0c, nt0, 0)
        hop1_round(e1r, e1c, nt1, 1)
        hop1_round(e2r, e2c, nt2, 2)

        def hop2_round(ntr, j):
            zero_ACC()
            plsc.subcore_barrier()
            if j == 0:
                rel_apply(e0r, e0c, h1h.at[0, k])
            elif j == 1:
                rel_apply(e1r, e1c, h1h.at[1, k])
            else:
                rel_apply(e0r, e0c, h1h.at[2, k])
                rel_apply(e1r, e1c, h1h.at[2, k])
                rel_apply(e2r, e2c, h1h.at[2, k])
            plsc.subcore_barrier()
            wa = wv[4 + j, pl.ds(0, 16)]
            wb = wv[4 + j, pl.ds(16, 16)]
            def vsub(sub, carry2):
                base = nb + sub * SUB
                pltpu.sync_copy(ACC.at[pl.ds(base, SUB)], P0.at[pl.ds(0, SUB)])
                if j > 0:
                    pltpu.sync_copy(uh.at[c, pl.ds(base, SUB)], P1.at[pl.ds(0, SUB)])
                def nbody(n, c3):
                    nv = splat(ntr, sub * SUB + n)
                    fa = P0[n, pl.ds(0, 16)] * nv * wa
                    fb = P0[n, pl.ds(16, 16)] * nv * wb
                    if j > 0:
                        fa = fa + P1[n, pl.ds(0, 16)]
                        fb = fb + P1[n, pl.ds(16, 16)]
                    R[n, pl.ds(0, 16)] = fa
                    R[n, pl.ds(16, 16)] = fb
                    return c3
                lax.fori_loop(0, SUB, nbody, 0)
                if j == 2:
                    pltpu.sync_copy(R.at[pl.ds(0, SUB)],
                                    out.at[2, k, pl.ds(base, SUB)])
                else:
                    pltpu.sync_copy(R.at[pl.ds(0, SUB)],
                                    uh.at[c, pl.ds(base, SUB)])
                return carry2
            lax.fori_loop(0, NT // SUB, vsub, 0)
            plsc.subcore_barrier()

        hop2_round(nt0, 0)
        hop2_round(nt1, 1)
        hop2_round(nt2, 2)

        w0a = wv[0, pl.ds(0, 16)]
        w0b = wv[0, pl.ds(16, 16)]
        w1 = [(wv[1 + r, pl.ds(0, 16)], wv[1 + r, pl.ds(16, 16)])
              for r in range(3)]
        def osub(sub, carry2):
            base = nb + sub * SUB
            pltpu.sync_copy(xs.at[k, pl.ds(base, SUB)], P0.at[pl.ds(0, SUB)])
            def o0(n, c3):
                R[n, pl.ds(0, 16)] = P0[n, pl.ds(0, 16)] * w0a
                R[n, pl.ds(16, 16)] = P0[n, pl.ds(16, 16)] * w0b
                return c3
            lax.fori_loop(0, SUB, o0, 0)
            pltpu.sync_copy(R.at[pl.ds(0, SUB)], out.at[0, k, pl.ds(base, SUB)])
            pltpu.sync_copy(h1h.at[0, k, pl.ds(base, SUB)], P0.at[pl.ds(0, SUB)])
            pltpu.sync_copy(h1h.at[1, k, pl.ds(base, SUB)], P1.at[pl.ds(0, SUB)])
            pltpu.sync_copy(h1h.at[2, k, pl.ds(base, SUB)], R2.at[pl.ds(0, SUB)])
            def o1(n, c3):
                for j in range(2):
                    sl = pl.ds(j * 16, 16)
                    R[n, sl] = (P0[n, sl] * w1[0][j]
                                + P1[n, sl] * w1[1][j]
                                + R2[n, sl] * w1[2][j])
                return c3
            lax.fori_loop(0, SUB, o1, 0)
            pltpu.sync_copy(R.at[pl.ds(0, SUB)], out.at[1, k, pl.ds(base, SUB)])
            return carry2
        lax.fori_loop(0, NT // SUB, osub, 0)
        plsc.subcore_barrier()
        return carry

    lax.fori_loop(0, 2, pass_body, 0)


_nars_sc = functools.partial(
    pl.kernel,
    out_type=(
        jax.ShapeDtypeStruct((3, NSLICE, NPAD, DS), F32),
        jax.ShapeDtypeStruct((3, NSLICE, NPAD, DS), F32),
        jax.ShapeDtypeStruct((2, 16, NPAD), F32),
        jax.ShapeDtypeStruct((2, NPAD, DS), F32),
    ),
    mesh=plsc.VectorSubcoreMesh(core_axis_name="c", subcore_axis_name="s"),
    compiler_params=pltpu.CompilerParams(
        needs_layout_passes=False, use_tc_tiling_on_sc=False),
    scratch_types=[
        pltpu.VMEM_SHARED((NPAD, DS), F32),
        pltpu.VMEM((NPAD,), F32),
        pltpu.VMEM((NEC, EC), I32),
        pltpu.VMEM((NEC, EC), I32),
        pltpu.VMEM((EC, DS), F32),
        pltpu.VMEM((EC, DS), F32),
        pltpu.VMEM((EC, DS), F32),
        pltpu.VMEM((EC, DS), F32),
        pltpu.VMEM((EC, DS), F32),
        pltpu.VMEM((EC, DS), F32),
        pltpu.VMEM((EC, DS), F32),
        pltpu.VMEM((EC, DS), F32),
        pltpu.VMEM((NT,), F32),
        pltpu.VMEM((NT,), F32),
        pltpu.VMEM((NT,), F32),
        pltpu.VMEM((NT,), F32),
        pltpu.VMEM((8, DS), F32),
        pltpu.SemaphoreType.DMA,
        pltpu.SemaphoreType.DMA,
        pltpu.SemaphoreType.DMA,
        pltpu.SemaphoreType.DMA,
        pltpu.SemaphoreType.DMA,
        pltpu.SemaphoreType.DMA,
        pltpu.SemaphoreType.DMA,
        pltpu.SemaphoreType.DMA,
    ],
)(_sc_body)


def kernel(x, edge_index_r0, edge_index_r1, edge_index_r2, weight):
    xp = jnp.pad(x, ((0, NPAD - N), (0, 0)))
    xs = xp.reshape(NPAD, NSLICE, DS).transpose(1, 0, 2)
    w = weight.reshape(3, 3, D)
    w0c = w[0].sum(axis=0)
    rows = jnp.concatenate([w0c[None], w[1], w[2], jnp.zeros((1, D), F32)],
                           axis=0)
    wts = rows.reshape(8, NSLICE, DS).transpose(1, 0, 2)
    e2d = [e[i].reshape(16 * NEC, EC)
           for e in (edge_index_r0, edge_index_r1, edge_index_r2)
           for i in (0, 1)]
    out4, _, _, _ = _nars_sc(xs, *e2d, wts)
    out = out4.transpose(0, 2, 1, 3).reshape(3, NPAD, D)[:, :N]
    return out
